# baseline (speedup 1.0000x reference)
import functools

import jax
import jax.numpy as jnp
from jax.experimental import pallas as pl

_DIM = 2048
_NE = 64
_K = 8
_BT = 1024
_S = 16
_SW = _DIM // _S


def _gate_block(*refs):
    x_refs = refs[:_S]
    w_ref = refs[_S]
    vals_ref, idx_ref = refs[_S + 1], refs[_S + 2]
    w = w_ref[...]
    logits = jnp.zeros((_BT, _NE), jnp.float32)
    for s in range(_S):
        logits += jax.lax.dot_general(
            x_refs[s][...],
            w[:, s * _SW : (s + 1) * _SW],
            (((1,), (1,)), ((), ())),
            preferred_element_type=jnp.float32,
        )
    rev_f = ((_NE - 1) - jax.lax.broadcasted_iota(jnp.int32, (_BT, _NE), 1)).astype(
        jnp.float32
    )
    neg_inf = jnp.float32(-jnp.inf)
    l = logits
    tops = []
    ridxs = []
    for _ in range(_K):
        m = jnp.max(l, axis=1, keepdims=True)
        r = jnp.max(jnp.where(l == m, rev_f, jnp.float32(-1.0)), axis=1, keepdims=True)
        l = jnp.where(rev_f == r, neg_inf, l)
        tops.append(m)
        ridxs.append(r)
    top_val = jnp.concatenate(tops, axis=1)
    top_idx = (_NE - 1) - jnp.concatenate(ridxs, axis=1).astype(jnp.int32)
    s = jax.nn.sigmoid(top_val)
    s = s / jnp.sum(s, axis=1, keepdims=True)
    vals_ref[...] = s
    idx_ref[...] = top_idx


@jax.jit
def kernel(x, weight):
    t = x.shape[0]
    grid = (t // _BT,)
    vals, idx = pl.pallas_call(
        _gate_block,
        grid=grid,
        in_specs=[
            pl.BlockSpec((_BT, _SW), functools.partial(lambda s, i: (i, s), s))
            for s in range(_S)
        ]
        + [
            pl.BlockSpec((_NE, _DIM), lambda i: (0, 0)),
        ],
        out_specs=[
            pl.BlockSpec((_BT, _K), lambda i: (i, 0)),
            pl.BlockSpec((_BT, _K), lambda i: (i, 0)),
        ],
        out_shape=[
            jax.ShapeDtypeStruct((t, _K), jnp.float32),
            jax.ShapeDtypeStruct((t, _K), jnp.int32),
        ],
    )(*([x] * _S), weight)
    return vals, idx


# speedup vs baseline: 1.1440x; 1.1440x over previous
import functools

import jax
import jax.numpy as jnp
from jax.experimental import pallas as pl

_DIM = 2048
_NE = 64
_K = 8
_BT = 1024
_S = 16
_SW = _DIM // _S


def _gate_block(*refs):
    x_refs = refs[:_S]
    w_ref = refs[_S]
    vals_ref, idx_ref = refs[_S + 1], refs[_S + 2]
    w = w_ref[...]
    lt = jnp.zeros((_NE, _BT), jnp.float32)
    for s in range(_S):
        lt += jax.lax.dot_general(
            w[:, s * _SW : (s + 1) * _SW],
            x_refs[s][...],
            (((1,), (1,)), ((), ())),
            preferred_element_type=jnp.float32,
        )
    rev_f = ((_NE - 1) - jax.lax.broadcasted_iota(jnp.int32, (_NE, _BT), 0)).astype(
        jnp.float32
    )
    neg_inf = jnp.float32(-jnp.inf)
    tops = []
    ridxs = []
    for _ in range(_K):
        m = jnp.max(lt, axis=0, keepdims=True)
        r = jnp.max(jnp.where(lt == m, rev_f, jnp.float32(-1.0)), axis=0, keepdims=True)
        lt = jnp.where(rev_f == r, neg_inf, lt)
        tops.append(m)
        ridxs.append(r)
    top_val = jnp.concatenate(tops, axis=0)
    top_idx = (_NE - 1) - jnp.concatenate(ridxs, axis=0).astype(jnp.int32)
    s = jax.nn.sigmoid(top_val)
    s = s / jnp.sum(s, axis=0, keepdims=True)
    vals_ref[...] = s.T
    idx_ref[...] = top_idx.T


@jax.jit
def kernel(x, weight):
    t = x.shape[0]
    grid = (t // _BT,)
    vals, idx = pl.pallas_call(
        _gate_block,
        grid=grid,
        in_specs=[
            pl.BlockSpec((_BT, _SW), functools.partial(lambda s, i: (i, s), s))
            for s in range(_S)
        ]
        + [
            pl.BlockSpec((_NE, _DIM), lambda i: (0, 0)),
        ],
        out_specs=[
            pl.BlockSpec((_BT, _K), lambda i: (i, 0)),
            pl.BlockSpec((_BT, _K), lambda i: (i, 0)),
        ],
        out_shape=[
            jax.ShapeDtypeStruct((t, _K), jnp.float32),
            jax.ShapeDtypeStruct((t, _K), jnp.int32),
        ],
    )(*([x] * _S), weight)
    return vals, idx


# speedup vs baseline: 1.1825x; 1.0337x over previous
import functools

import jax
import jax.numpy as jnp
from jax.experimental import pallas as pl

_DIM = 2048
_NE = 64
_K = 8
_BT = 1024
_S = 8
_SB = _BT // _S


def _gate_block(*refs):
    x_refs = refs[:_S]
    w_ref = refs[_S]
    vals_ref, idx_ref = refs[_S + 1], refs[_S + 2]
    w = w_ref[...]
    lt = jnp.concatenate(
        [
            jax.lax.dot_general(
                w,
                x_refs[s][...],
                (((1,), (1,)), ((), ())),
                preferred_element_type=jnp.float32,
            )
            for s in range(_S)
        ],
        axis=1,
    )
    rev_f = ((_NE - 1) - jax.lax.broadcasted_iota(jnp.int32, (_NE, _BT), 0)).astype(
        jnp.float32
    )
    neg_inf = jnp.float32(-jnp.inf)
    tops = []
    ridxs = []
    for _ in range(_K):
        m = jnp.max(lt, axis=0, keepdims=True)
        r = jnp.max(jnp.where(lt == m, rev_f, jnp.float32(-1.0)), axis=0, keepdims=True)
        lt = jnp.where(rev_f == r, neg_inf, lt)
        tops.append(m)
        ridxs.append(r)
    top_val = jnp.concatenate(tops, axis=0)
    top_idx = (_NE - 1) - jnp.concatenate(ridxs, axis=0).astype(jnp.int32)
    s = jax.nn.sigmoid(top_val)
    s = s / jnp.sum(s, axis=0, keepdims=True)
    vals_ref[...] = s.T
    idx_ref[...] = top_idx.T


@jax.jit
def kernel(x, weight):
    t = x.shape[0]
    grid = (t // _BT,)
    vals, idx = pl.pallas_call(
        _gate_block,
        grid=grid,
        in_specs=[
            pl.BlockSpec(
                (_SB, _DIM), functools.partial(lambda s, i: (i * _S + s, 0), s)
            )
            for s in range(_S)
        ]
        + [
            pl.BlockSpec((_NE, _DIM), lambda i: (0, 0)),
        ],
        out_specs=[
            pl.BlockSpec((_BT, _K), lambda i: (i, 0)),
            pl.BlockSpec((_BT, _K), lambda i: (i, 0)),
        ],
        out_shape=[
            jax.ShapeDtypeStruct((t, _K), jnp.float32),
            jax.ShapeDtypeStruct((t, _K), jnp.int32),
        ],
    )(*([x] * _S), weight)
    return vals, idx


# speedup vs baseline: 1.5996x; 1.3527x over previous
import functools

import jax
import jax.numpy as jnp
from jax.experimental import pallas as pl

_DIM = 2048
_NE = 64
_K = 8
_BT = 1024
_S = 8
_SB = _BT // _S


def _gate_block(*refs):
    x_refs = refs[:_S]
    w_ref = refs[_S]
    vals_ref, idx_ref = refs[_S + 1], refs[_S + 2]
    w = w_ref[...]
    lt = jnp.concatenate(
        [
            jax.lax.dot_general(
                w,
                x_refs[s][...],
                (((1,), (1,)), ((), ())),
                preferred_element_type=jnp.float32,
            )
            for s in range(_S)
        ],
        axis=1,
    )
    rev_f = ((_NE - 1) - jax.lax.broadcasted_iota(jnp.int32, (_NE, _BT), 0)).astype(
        jnp.float32
    )
    neg_inf = jnp.float32(-jnp.inf)
    tops = []
    ridxs = []
    for _ in range(_K):
        m = jnp.max(lt, axis=0, keepdims=True)
        r = jnp.max(jnp.where(lt == m, rev_f, jnp.float32(-1.0)), axis=0, keepdims=True)
        lt = jnp.where(rev_f == r, neg_inf, lt)
        tops.append(m)
        ridxs.append(r)
    top_val = jnp.concatenate(tops, axis=0)
    top_idx = (_NE - 1) - jnp.concatenate(ridxs, axis=0).astype(jnp.int32)
    s = jax.nn.sigmoid(top_val)
    s = s / jnp.sum(s, axis=0, keepdims=True)
    vals_ref[...] = s
    idx_ref[...] = top_idx


@jax.jit
def kernel(x, weight):
    t = x.shape[0]
    grid = (t // _BT,)
    vals, idx = pl.pallas_call(
        _gate_block,
        grid=grid,
        in_specs=[
            pl.BlockSpec(
                (_SB, _DIM), functools.partial(lambda s, i: (i * _S + s, 0), s)
            )
            for s in range(_S)
        ]
        + [
            pl.BlockSpec((_NE, _DIM), lambda i: (0, 0)),
        ],
        out_specs=[
            pl.BlockSpec((_K, _BT), lambda i: (0, i)),
            pl.BlockSpec((_K, _BT), lambda i: (0, i)),
        ],
        out_shape=[
            jax.ShapeDtypeStruct((_K, t), jnp.float32),
            jax.ShapeDtypeStruct((_K, t), jnp.int32),
        ],
    )(*([x] * _S), weight)
    return vals.T, idx.T


# speedup vs baseline: 1.6422x; 1.0266x over previous
import functools

import jax
import jax.numpy as jnp
from jax.experimental import pallas as pl

_DIM = 2048
_NE = 64
_K = 8
_BT = 2048
_S = 16
_SB = _BT // _S


def _gate_block(*refs):
    x_refs = refs[:_S]
    w_ref = refs[_S]
    vals_ref, idx_ref = refs[_S + 1], refs[_S + 2]
    w = w_ref[...]
    lt = jnp.concatenate(
        [
            jax.lax.dot_general(
                w,
                x_refs[s][...],
                (((1,), (1,)), ((), ())),
                preferred_element_type=jnp.float32,
            )
            for s in range(_S)
        ],
        axis=1,
    )
    rev_f = ((_NE - 1) - jax.lax.broadcasted_iota(jnp.int32, (_NE, _BT), 0)).astype(
        jnp.float32
    )
    neg_inf = jnp.float32(-jnp.inf)
    tops = []
    ridxs = []
    for _ in range(_K):
        m = jnp.max(lt, axis=0, keepdims=True)
        r = jnp.max(jnp.where(lt == m, rev_f, jnp.float32(-1.0)), axis=0, keepdims=True)
        lt = jnp.where(rev_f == r, neg_inf, lt)
        tops.append(m)
        ridxs.append(r)
    top_val = jnp.concatenate(tops, axis=0)
    top_idx = (_NE - 1) - jnp.concatenate(ridxs, axis=0).astype(jnp.int32)
    s = jax.nn.sigmoid(top_val)
    s = s / jnp.sum(s, axis=0, keepdims=True)
    vals_ref[...] = s
    idx_ref[...] = top_idx


@jax.jit
def kernel(x, weight):
    t = x.shape[0]
    grid = (t // _BT,)
    vals, idx = pl.pallas_call(
        _gate_block,
        grid=grid,
        in_specs=[
            pl.BlockSpec(
                (_SB, _DIM), functools.partial(lambda s, i: (i * _S + s, 0), s)
            )
            for s in range(_S)
        ]
        + [
            pl.BlockSpec((_NE, _DIM), lambda i: (0, 0)),
        ],
        out_specs=[
            pl.BlockSpec((_K, _BT), lambda i: (0, i)),
            pl.BlockSpec((_K, _BT), lambda i: (0, i)),
        ],
        out_shape=[
            jax.ShapeDtypeStruct((_K, t), jnp.float32),
            jax.ShapeDtypeStruct((_K, t), jnp.int32),
        ],
    )(*([x] * _S), weight)
    return vals.T, idx.T


# speedup vs baseline: 1.6861x; 1.0268x over previous
import functools

import jax
import jax.numpy as jnp
from jax.experimental import pallas as pl

_DIM = 2048
_NE = 64
_K = 8
_BT = 2048
_S = 16
_SB = _BT // _S


def _gate_block(*refs):
    x_refs = refs[:_S]
    w_ref = refs[_S]
    vals_ref, idx_ref = refs[_S + 1], refs[_S + 2]
    w = w_ref[...]
    rev_f = ((_NE - 1) - jax.lax.broadcasted_iota(jnp.int32, (_NE, _SB), 0)).astype(
        jnp.float32
    )
    neg_inf = jnp.float32(-jnp.inf)
    for sb in range(_S):
        lt = jax.lax.dot_general(
            w,
            x_refs[sb][...],
            (((1,), (1,)), ((), ())),
            preferred_element_type=jnp.float32,
        )
        tops = []
        ridxs = []
        for _ in range(_K):
            m = jnp.max(lt, axis=0, keepdims=True)
            r = jnp.max(
                jnp.where(lt == m, rev_f, jnp.float32(-1.0)), axis=0, keepdims=True
            )
            lt = jnp.where(rev_f == r, neg_inf, lt)
            tops.append(m)
            ridxs.append(r)
        top_val = jnp.concatenate(tops, axis=0)
        top_idx = (_NE - 1) - jnp.concatenate(ridxs, axis=0).astype(jnp.int32)
        s = jax.nn.sigmoid(top_val)
        s = s / jnp.sum(s, axis=0, keepdims=True)
        cols = pl.ds(sb * _SB, _SB)
        vals_ref[:, cols] = s
        idx_ref[:, cols] = top_idx


@jax.jit
def kernel(x, weight):
    t = x.shape[0]
    grid = (t // _BT,)
    vals, idx = pl.pallas_call(
        _gate_block,
        grid=grid,
        in_specs=[
            pl.BlockSpec(
                (_SB, _DIM), functools.partial(lambda s, i: (i * _S + s, 0), s)
            )
            for s in range(_S)
        ]
        + [
            pl.BlockSpec((_NE, _DIM), lambda i: (0, 0)),
        ],
        out_specs=[
            pl.BlockSpec((_K, _BT), lambda i: (0, i)),
            pl.BlockSpec((_K, _BT), lambda i: (0, i)),
        ],
        out_shape=[
            jax.ShapeDtypeStruct((_K, t), jnp.float32),
            jax.ShapeDtypeStruct((_K, t), jnp.int32),
        ],
    )(*([x] * _S), weight)
    return vals.T, idx.T
